# baseline (speedup 1.0000x reference)
import jax
import jax.numpy as jnp
from jax import lax
from jax.experimental import pallas as pl
from jax.experimental.pallas import tpu as pltpu

_NC = 100
_CW = 1024
_NTOT = _NC * _CW
_TOPK = 1000
_NCOL = 8


def _body(data_ref, sc_ref, out_lvl_ref, out_res_ref, out_cnt_ref,
          dest_ref, sel_ref):
    f32 = jnp.float32
    scores = sc_ref[...]
    keys = lax.bitcast_convert_type(scores, jnp.int32)

    def bs(_, lohi):
        lo, hi = lohi
        mid = lo + (hi - lo) // 2
        c = jnp.sum((keys >= mid).astype(jnp.int32))
        big = c >= _TOPK
        return jnp.where(big, mid, lo), jnp.where(big, hi, mid)

    lo, _ = lax.fori_loop(0, 31, bs, (jnp.int32(0), jnp.int32(1 << 30)))
    T = lo
    cnt_gt = jnp.sum((keys > T).astype(jnp.int32))
    need_eq = (_TOPK - cnt_gt).astype(f32)

    i0 = lax.broadcasted_iota(jnp.int32, (_CW, _CW), 0)
    i1 = lax.broadcasted_iota(jnp.int32, (_CW, _CW), 1)
    W = (i0 < i1).astype(f32)
    eye = (i0 == i1).astype(f32)
    r0 = lax.broadcasted_iota(jnp.int32, (_NC, _NC), 0)
    r1 = lax.broadcasted_iota(jnp.int32, (_NC, _NC), 1)
    Mrows = (r1 < r0).astype(f32)
    eye_r = (r0 == r1).astype(f32)

    def exprefix(x):
        inrow = jnp.dot(x, W, preferred_element_type=f32)
        rows_col = jnp.sum(x, axis=1, keepdims=True)
        rows_row = jnp.sum(eye_r * rows_col, axis=0, keepdims=True)
        pre = jnp.sum(Mrows * rows_row, axis=1, keepdims=True)
        return inrow + pre

    eqf = (keys == T).astype(f32)
    eqrank = exprefix(eqf)
    sel = jnp.where((keys > T) | ((keys == T) & (eqrank < need_eq)), 1.0, 0.0)
    dest_ref[...] = exprefix(sel)
    sel_ref[...] = sel

    d_iota = i0.astype(f32)

    def chunk(c, acc):
        dr = dest_ref[pl.ds(c, 1), :]
        sr = sel_ref[pl.ds(c, 1), :]
        oh = (d_iota == dr).astype(f32) * sr
        blk = data_ref[pl.ds(c * _CW, _CW), :]
        return acc + jnp.dot(oh, blk, preferred_element_type=f32,
                             precision=lax.Precision.HIGHEST)

    compact = lax.fori_loop(0, _NC, chunk, jnp.zeros((_CW, _NCOL), f32))

    col_iota = lax.broadcasted_iota(jnp.int32, (_CW, 1), 0).astype(f32)
    row_iota = lax.broadcasted_iota(jnp.int32, (1, _CW), 1).astype(f32)
    padmask_col = col_iota < float(_TOPK)

    def tr(v_col):
        return jnp.sum(eye * v_col, axis=0, keepdims=True)

    sc_col = jnp.where(padmask_col, compact[:, 5:6], -1.0)
    ix_col = compact[:, 6:7]
    sc_row = tr(sc_col)
    ix_row = tr(ix_col)
    P = (sc_col > sc_row) | ((sc_col == sc_row) & (ix_col < ix_row))
    rank_row = jnp.sum(P.astype(f32), axis=0, keepdims=True)
    ohT = (col_iota == rank_row).astype(f32)
    sorted_d = jnp.dot(ohT, compact, preferred_element_type=f32,
                       precision=lax.Precision.HIGHEST)

    w = sorted_d[:, 3:4] - sorted_d[:, 1:2] + 1.0
    h = sorted_d[:, 4:5] - sorted_d[:, 2:3] + 1.0
    s = jnp.sqrt(w * h)
    lvl_col = jnp.floor(4.0 + jnp.log2(s / 224.0 + 1e-6))
    lvl_col = jnp.clip(lvl_col, 2.0, 5.0)
    lvl_col = jnp.where(padmask_col, lvl_col, 6.0)
    lvl_row = tr(lvl_col)
    P2 = (lvl_col < lvl_row) | ((lvl_col == lvl_row) & (col_iota < row_iota))
    rank2_row = jnp.sum(P2.astype(f32), axis=0, keepdims=True)
    ohT2 = (col_iota == rank2_row).astype(f32)
    out_lvl_ref[...] = jnp.dot(ohT2, sorted_d, preferred_element_type=f32,
                               precision=lax.Precision.HIGHEST)
    out_res_ref[...] = rank2_row.astype(jnp.int32)
    cnt_iota = lax.broadcasted_iota(jnp.int32, (1, 128), 1).astype(f32) + 2.0
    C = (lvl_col == cnt_iota).astype(f32)
    out_cnt_ref[...] = jnp.sum(C, axis=0, keepdims=True).astype(jnp.int32)


def _call(data, scp):
    return pl.pallas_call(
        _body,
        out_shape=[
            jax.ShapeDtypeStruct((_CW, _NCOL), jnp.float32),
            jax.ShapeDtypeStruct((1, _CW), jnp.int32),
            jax.ShapeDtypeStruct((1, 128), jnp.int32),
        ],
        scratch_shapes=[
            pltpu.VMEM((_NC, _CW), jnp.float32),
            pltpu.VMEM((_NC, _CW), jnp.float32),
        ],
    )(data, scp)


def kernel(rpn_rois_fpn2, rpn_rois_fpn3, rpn_rois_fpn4, rpn_rois_fpn5,
           rpn_rois_fpn6, rpn_roi_probs_fpn2, rpn_roi_probs_fpn3,
           rpn_roi_probs_fpn4, rpn_roi_probs_fpn5, rpn_roi_probs_fpn6,
           im_info, roidb):
    rois = jnp.concatenate([rpn_rois_fpn2, rpn_rois_fpn3, rpn_rois_fpn4,
                            rpn_rois_fpn5, rpn_rois_fpn6], axis=0)
    sc = jnp.concatenate([rpn_roi_probs_fpn2, rpn_roi_probs_fpn3,
                          rpn_roi_probs_fpn4, rpn_roi_probs_fpn5,
                          rpn_roi_probs_fpn6], axis=0)[:, 0]
    n = rois.shape[0]
    idx = jnp.arange(n, dtype=jnp.float32)[:, None]
    data = jnp.concatenate(
        [rois, sc[:, None], idx, jnp.zeros((n, 1), jnp.float32)], axis=1)
    data = jnp.pad(data, ((0, _NTOT - n), (0, 0)))
    scp = jnp.pad(sc, (0, _NTOT - n), constant_values=-1.0).reshape(_NC, _CW)
    lvl_sorted, restore, cnts = _call(data, scp)
    rois_by_level = lvl_sorted[:_TOPK, :5]
    rois_idx_restore = restore.reshape(_CW)[:_TOPK]
    lvl_counts = cnts.reshape(128)[:4]
    return rois_by_level, rois_idx_restore, lvl_counts
